# baseline (speedup 1.0000x reference)
import functools

import jax
import jax.numpy as jnp
from jax import lax
from jax.experimental import pallas as pl
from jax.experimental.pallas import tpu as pltpu
from jax.experimental.pallas import tpu_sc as plsc

N0 = 40962
E0 = 245760
N1 = 10242
E1 = 61440
K = 3

NP0 = 43008
NP1 = 12288
NC = 2
NS = 16
NW = NC * NS
BM = 256

CAP0 = 8960
CAP1 = 2560

_f32 = jnp.float32
_i32 = jnp.int32


def _chunks(total):
    if total < 128:
        assert total % 8 == 0
        return [(0, total)]
    out = [(b, 128) for b in range(0, total - 127, 128)]
    if total % 128:
        out.append((total - 128, 128))
    return out


def _sc_mesh():
    return plsc.VectorSubcoreMesh(core_axis_name="c", subcore_axis_name="s",
                                  num_cores=NC, num_subcores=NS)


_SC_PARAMS = pltpu.CompilerParams(needs_layout_passes=False)


def _route_body(e, cap, rng,
                dst2d, src2d, px2d, py2d, z128,
                srcr, dlr, pxr, pyr, deg,
                dbd, dbs, dbx, dby, srcb, dlb, pxb, pyb, degacc):
    c = lax.axis_index("c")
    s = lax.axis_index("s")
    wid = c * NS + s
    lo = wid * rng
    iota = lax.iota(_i32, 16)
    ones = jnp.ones((16,), _f32)
    lane0 = iota < 1
    zero16 = jnp.full((16,), 0, _i32)
    zf = jnp.zeros((16,), _f32)
    nch = cap // 128

    def pre(i, _):
        r = lax.shift_right_logical(i, 3)
        sl = pl.ds((i & 7) * 16, 16)
        srcb[r, sl] = zero16
        dlb[r, sl] = jnp.full((16,), rng, _i32)
        pxb[r, sl] = zf
        pyb[r, sl] = zf
        return 0

    lax.fori_loop(0, nch * 8, pre, 0)
    for b, sz in _chunks(rng // 8 + 8):
        pltpu.sync_copy(z128.at[pl.ds(0, sz)], degacc.at[pl.ds(b, sz)])

    def macro(mc, off):
        pltpu.sync_copy(dst2d.at[pl.ds(mc * 8, 8)], dbd)
        pltpu.sync_copy(src2d.at[pl.ds(mc * 8, 8)], dbs)
        pltpu.sync_copy(px2d.at[pl.ds(mc * 8, 8)], dbx)
        pltpu.sync_copy(py2d.at[pl.ds(mc * 8, 8)], dby)
        for r in range(8):
            for l in range(8):
                sl = pl.ds(l * 16, 16)
                dl = dbd[r, sl] - lo
                m = (dl >= 0) & (dl < rng)
                mi = m.astype(_i32)
                idx = jnp.minimum(off + plsc.cumsum(mi) - 1, cap - 1)
                row = lax.shift_right_logical(idx, 7)
                col = idx & 127
                plsc.store_scatter(dlb, [row, col], dl, mask=m)
                plsc.store_scatter(srcb, [row, col], dbs[r, sl], mask=m)
                plsc.store_scatter(pxb, [row, col], dbx[r, sl], mask=m)
                plsc.store_scatter(pyb, [row, col], dby[r, sl], mask=m)
                off = off + jnp.sum(mi)
        return off

    lax.fori_loop(0, e // 1024, macro, jnp.int32(0))

    def chunk(cj, _):
        def edge(ei, _):
            dls = plsc.load_gather(dlb, [zero16 + cj, zero16 + ei])
            plsc.addupdate_scatter(
                degacc,
                [lax.shift_right_logical(dls, 3), (dls & 7) * 16 + iota],
                ones, mask=lane0)
            return 0

        lax.fori_loop(0, 128, edge, 0)
        return 0

    lax.fori_loop(0, nch, chunk, 0)

    pltpu.sync_copy(srcb, srcr.at[wid])
    pltpu.sync_copy(dlb, dlr.at[wid])
    pltpu.sync_copy(pxb, pxr.at[wid])
    pltpu.sync_copy(pyb, pyr.at[wid])
    pltpu.sync_copy(degacc.at[pl.ds(0, rng // 8)],
                    deg.at[pl.ds(wid * (rng // 8), rng // 8)])


def _make_route_kernel(e, np_, cap):
    rng = np_ // NW
    body = functools.partial(_route_body, e, cap, rng)
    slab_i = jax.ShapeDtypeStruct((NW, cap // 128, 128), _i32)
    slab_f = jax.ShapeDtypeStruct((NW, cap // 128, 128), _f32)
    return pl.kernel(
        body,
        out_type=(slab_i, slab_i, slab_f, slab_f,
                  jax.ShapeDtypeStruct((np_ // 8, 128), _f32)),
        mesh=_sc_mesh(),
        scratch_types=[
            pltpu.VMEM((8, 128), _i32),
            pltpu.VMEM((8, 128), _i32),
            pltpu.VMEM((8, 128), _f32),
            pltpu.VMEM((8, 128), _f32),
            pltpu.VMEM((cap // 128, 128), _i32),
            pltpu.VMEM((cap // 128, 128), _i32),
            pltpu.VMEM((cap // 128, 128), _f32),
            pltpu.VMEM((cap // 128, 128), _f32),
            pltpu.VMEM((rng // 8 + 8, 128), _f32),
        ],
        compiler_params=_SC_PARAMS,
    )


def _edge_body(cap, rng, fout,
               xg, srcr, dlr, pxr, pyr, consts, zacc, out,
               src_s, dl_s, px_s, py_s, w_v, cv, xj, acc, sem):
    c = lax.axis_index("c")
    s = lax.axis_index("s")
    wid = c * NS + s
    iota = lax.iota(_i32, 16)
    zero16 = jnp.full((16,), 0, _i32)

    p = 128 // fout
    psh = p.bit_length() - 1

    pltpu.sync_copy(srcr.at[wid], src_s)
    pltpu.sync_copy(dlr.at[wid], dl_s)
    pltpu.sync_copy(pxr.at[wid], px_s)
    pltpu.sync_copy(pyr.at[wid], py_s)
    pltpu.sync_copy(consts, cv)
    for b, sz in _chunks(rng // p + 8):
        pltpu.sync_copy(zacc.at[pl.ds(0, sz)], acc.at[pl.ds(b, sz)])

    def chunk(cj, _):
        pltpu.async_copy(xg.at[src_s.at[cj]], xj, sem).wait()

        def group(g, _):
            px = px_s[cj, pl.ds(g * 16, 16)]
            py = py_s[cj, pl.ds(g * 16, 16)]
            for k in range(K):
                dx = px - cv[k, :]
                dy = py - cv[K + k, :]
                w_v[k, pl.ds(g * 16, 16)] = jnp.exp(
                    dx * dx * cv[2 * K + k, :] + dy * dy * cv[3 * K + k, :])
            return 0

        lax.fori_loop(0, 8, group, 0)

        def edge(ei, _):
            dls = plsc.load_gather(dl_s, [zero16 + cj, zero16 + ei])
            w0 = plsc.load_gather(w_v, [zero16, zero16 + ei])
            w1 = plsc.load_gather(w_v, [zero16 + 1, zero16 + ei])
            w2 = plsc.load_gather(w_v, [zero16 + 2, zero16 + ei])
            row = lax.shift_right_logical(dls, psh)
            lbase = (dls & (p - 1)) * fout
            for fb in range(fout // 16):
                sl = fb * 16
                m = (xj[ei, pl.ds(sl, 16)] * w0
                     + xj[ei, pl.ds(fout + sl, 16)] * w1
                     + xj[ei, pl.ds(2 * fout + sl, 16)] * w2)
                plsc.addupdate_scatter(acc, [row, lbase + sl + iota], m)
            return 0

        lax.fori_loop(0, 128, edge, 0)
        return 0

    lax.fori_loop(0, cap // 128, chunk, 0)

    pltpu.sync_copy(acc.at[pl.ds(0, rng // p)],
                    out.at[pl.ds(wid * (rng // p), rng // p)])


def _make_edge_kernel(np_, cap, fout):
    rng = np_ // NW
    p = 128 // fout
    body = functools.partial(_edge_body, cap, rng, fout)
    return pl.kernel(
        body,
        out_type=jax.ShapeDtypeStruct((np_ // p, 128), _f32),
        mesh=_sc_mesh(),
        scratch_types=[
            pltpu.VMEM((cap // 128, 128), _i32),
            pltpu.VMEM((cap // 128, 128), _i32),
            pltpu.VMEM((cap // 128, 128), _f32),
            pltpu.VMEM((cap // 128, 128), _f32),
            pltpu.VMEM((K, 128), _f32),
            pltpu.VMEM((4 * K, 16), _f32),
            pltpu.VMEM((128, (K + 1) * fout), _f32),
            pltpu.VMEM((rng // p + 8, 128), _f32),
            pltpu.SemaphoreType.DMA,
        ],
        compiler_params=_SC_PARAMS,
    )


def _pool_body(x, nbr2d, out, nbr_v, bufa, bufb, sem):
    c = lax.axis_index("c")
    s = lax.axis_index("s")
    wid = c * NS + s
    nch = NP1 // 128
    iota = lax.iota(_i32, 16)
    seven = jnp.full((16,), 7, _i32)

    for t in range((nch + NW - 1) // NW):
        ch = wid + t * NW

        @pl.when(ch < nch)
        def _():
            ridx = lax.rem(iota, seven) * nch + ch
            pltpu.async_copy(nbr2d.at[ridx], nbr_v, sem).wait()
            pltpu.async_copy(x.at[nbr_v.at[0]], bufa, sem).wait()

            def fold(jj):
                pltpu.async_copy(x.at[nbr_v.at[jj]], bufb, sem).wait()

                def rstep(r, _):
                    for fb in range(2):
                        sl = pl.ds(fb * 16, 16)
                        bufa[r, sl] = jnp.maximum(bufa[r, sl], bufb[r, sl])
                    return 0

                lax.fori_loop(0, 128, rstep, 0)

            for jj in range(1, 7):
                fold(jj)
            pltpu.sync_copy(bufa, out.at[pl.ds(ch * 128, 128)])


def _make_pool_kernel():
    return pl.kernel(
        _pool_body,
        out_type=jax.ShapeDtypeStruct((NP1, 128), _f32),
        mesh=_sc_mesh(),
        scratch_types=[
            pltpu.VMEM((16, 128), _i32),
            pltpu.VMEM((128, 128), _f32),
            pltpu.VMEM((128, 128), _f32),
            pltpu.SemaphoreType.DMA,
        ],
        compiler_params=_SC_PARAMS,
    )


def _unpool_body(x1, ups2d, rest, uv, bufa, bufb, sem):
    c = lax.axis_index("c")
    s = lax.axis_index("s")
    wid = c * NS + s
    nmean = (N0 - N1) // 128
    iota = lax.iota(_i32, 16)

    for t in range((nmean + NW - 1) // NW):
        ch = wid + t * NW

        @pl.when(ch < nmean)
        def _():
            ridx = (iota & 1) * nmean + ch
            pltpu.async_copy(ups2d.at[ridx], uv, sem).wait()
            pltpu.async_copy(x1.at[uv.at[0]], bufa, sem).wait()
            pltpu.async_copy(x1.at[uv.at[1]], bufb, sem).wait()

            def rstep(r, _):
                for fb in range(4):
                    sl = pl.ds(fb * 16, 16)
                    bufa[r, sl] = 0.5 * (bufa[r, sl] + bufb[r, sl])
                return 0

            lax.fori_loop(0, 128, rstep, 0)
            pltpu.sync_copy(bufa, rest.at[pl.ds(ch * 128, 128)])


def _make_unpool_kernel():
    return pl.kernel(
        _unpool_body,
        out_type=jax.ShapeDtypeStruct((N0 - N1, 128), _f32),
        mesh=_sc_mesh(),
        scratch_types=[
            pltpu.VMEM((16, 128), _i32),
            pltpu.VMEM((128, 128), _f32),
            pltpu.VMEM((128, 128), _f32),
            pltpu.SemaphoreType.DMA,
        ],
        compiler_params=_SC_PARAMS,
    )


def _mm_body(x_ref, w_ref, o_ref):
    o_ref[...] = jnp.dot(x_ref[...], w_ref[...], preferred_element_type=_f32)


def _row_pad(w):
    return jnp.pad(w, ((0, 128 - w.shape[0]), (0, 0)))


def _conv_mm(x, g, root):
    np_ = x.shape[0]
    w = _row_pad(jnp.concatenate([g, root], axis=1))
    kf = w.shape[1]
    return pl.pallas_call(
        _mm_body,
        grid=(np_ // BM,),
        in_specs=[
            pl.BlockSpec((BM, 128), lambda i: (i, 0)),
            pl.BlockSpec((128, kf), lambda i: (0, 0)),
        ],
        out_specs=pl.BlockSpec((BM, kf), lambda i: (i, 0)),
        out_shape=jax.ShapeDtypeStruct((np_, kf), _f32),
    )(x, w)


def _mm2_body(x1_ref, x2_ref, w1_ref, w2_ref, o_ref):
    o_ref[...] = (jnp.dot(x1_ref[...], w1_ref[...], preferred_element_type=_f32)
                  + jnp.dot(x2_ref[...], w2_ref[...],
                            preferred_element_type=_f32))


def _conv_mm2(x1, x2, g, root, f1):
    np_ = x1.shape[0]
    w = jnp.concatenate([g, root], axis=1)
    kf = w.shape[1]
    w1 = _row_pad(w[:f1])
    w2 = _row_pad(w[f1:])
    return pl.pallas_call(
        _mm2_body,
        grid=(np_ // BM,),
        in_specs=[
            pl.BlockSpec((BM, 128), lambda i: (i, 0)),
            pl.BlockSpec((BM, 128), lambda i: (i, 0)),
            pl.BlockSpec((128, kf), lambda i: (0, 0)),
            pl.BlockSpec((128, kf), lambda i: (0, 0)),
        ],
        out_specs=pl.BlockSpec((BM, kf), lambda i: (i, 0)),
        out_shape=jax.ShapeDtypeStruct((np_, kf), _f32),
    )(x1, x2, w1, w2)


def _epi_body(f, agg_ref, d_ref, xgr_ref, b_ref, o_ref):
    deg = jnp.maximum(d_ref[:, 0:1], 1.0)
    xr = xgr_ref[:, K * f:] + b_ref[0, :][None, :]
    res = jnp.maximum(agg_ref[...] / deg + xr, 0.0)
    o_ref[...] = jnp.pad(res, ((0, 0), (0, 128 - f)))


def _epilogue(agg, deg, xgr, bias):
    np_ = xgr.shape[0]
    f = bias.shape[0]
    b8 = jnp.tile(bias[None, :], (8, 1))
    return pl.pallas_call(
        functools.partial(_epi_body, f),
        grid=(np_ // BM,),
        in_specs=[
            pl.BlockSpec((BM, f), lambda i: (i, 0)),
            pl.BlockSpec((BM, 16), lambda i: (i, 0)),
            pl.BlockSpec((BM, (K + 1) * f), lambda i: (i, 0)),
            pl.BlockSpec((8, f), lambda i: (0, 0)),
        ],
        out_specs=pl.BlockSpec((BM, 128), lambda i: (i, 0)),
        out_shape=jax.ShapeDtypeStruct((np_, 128), _f32),
    )(agg, deg, xgr, b8)


def _fc_body(x_ref, w_ref, b_ref, o_ref):
    y = (jnp.dot(x_ref[...], w_ref[...], preferred_element_type=_f32)
         + b_ref[0, :][None, :])
    l0 = y[:, 0:1]
    l1 = y[:, 1:2]
    m = jnp.maximum(l0, l1)
    lse = m + jnp.log(jnp.exp(l0 - m) + jnp.exp(l1 - m))
    o_ref[...] = jnp.concatenate([l0 - lse, l1 - lse], axis=1)


def _fc_head(x, w, b):
    np_ = x.shape[0]
    wp = jnp.zeros((128, 128), _f32).at[:32, :2].set(w)
    bp = jnp.zeros((8, 128), _f32).at[0, :2].set(b)
    return pl.pallas_call(
        _fc_body,
        grid=(np_ // BM,),
        in_specs=[
            pl.BlockSpec((BM, 128), lambda i: (i, 0)),
            pl.BlockSpec((128, 128), lambda i: (0, 0)),
            pl.BlockSpec((8, 128), lambda i: (0, 0)),
        ],
        out_specs=pl.BlockSpec((BM, 2), lambda i: (i, 0)),
        out_shape=jax.ShapeDtypeStruct((np_, 2), _f32),
    )(x, wp, bp)


def _consts_for(p):
    mu = p["mu"].astype(_f32)
    sig = p["sigma"].astype(_f32)
    cc = -0.5 / (1e-14 + sig * sig)
    v = jnp.concatenate([mu[:, 0], mu[:, 1], cc[:, 0], cc[:, 1]])
    return jnp.tile(v[:, None], (1, 16))


def _edge_views(edge_index, pseudo):
    e = edge_index.shape[1]
    src2d = edge_index[0].astype(_i32).reshape(e // 128, 128)
    dst2d = edge_index[1].astype(_i32).reshape(e // 128, 128)
    px2d = pseudo[:, 0].astype(_f32).reshape(e // 128, 128)
    py2d = pseudo[:, 1].astype(_f32).reshape(e // 128, 128)
    return dst2d, src2d, px2d, py2d


def _pad_rows(a, n):
    return jnp.pad(a, ((0, n - a.shape[0]), (0, 0)))


def kernel(data, params, edge_index0, pseudo0, edge_index1, pseudo1,
           neigh_indices, upsample_indices):
    p = params
    x0 = _pad_rows(data.astype(_f32), NP0)

    ev0 = _edge_views(edge_index0, pseudo0)
    ev1 = _edge_views(edge_index1, pseudo1)

    nbr2d = jnp.pad(neigh_indices.astype(_i32), ((0, NP1 - N1), (0, 0)))
    nbr2d = nbr2d.T.reshape(7 * (NP1 // 128), 128)
    ups2d = upsample_indices.astype(_i32).T.reshape(2 * ((N0 - N1) // 128), 128)

    z128 = jnp.zeros((128, 128), _f32)

    srcr0, dlr0, pxr0, pyr0, deg0p = _make_route_kernel(E0, NP0, CAP0)(
        *ev0, z128)
    srcr1, dlr1, pxr1, pyr1, deg1p = _make_route_kernel(E1, NP1, CAP1)(
        *ev1, z128)
    deg0 = deg0p.reshape(NP0, 16)
    deg1 = deg1p.reshape(NP1, 16)

    edge0 = _make_edge_kernel(NP0, CAP0, 32)
    edge1 = _make_edge_kernel(NP1, CAP1, 64)

    def conv0(xgr, cp):
        agg = edge0(xgr, srcr0, dlr0, pxr0, pyr0,
                    _consts_for(cp), z128).reshape(NP0, 32)
        return _epilogue(agg, deg0, xgr, cp["bias"])

    def conv1(xgr, cp):
        agg = edge1(xgr, srcr1, dlr1, pxr1, pyr1,
                    _consts_for(cp), z128).reshape(NP1, 64)
        return _epilogue(agg, deg1, xgr, cp["bias"])

    x = conv0(_conv_mm(x0, p["e00"]["g"], p["e00"]["root"]), p["e00"])
    x = conv0(_conv_mm(x, p["e01"]["g"], p["e01"]["root"]), p["e01"])
    skip0 = x

    x1 = _make_pool_kernel()(x, nbr2d)
    x1 = conv1(_conv_mm(x1, p["e10"]["g"], p["e10"]["root"]), p["e10"])
    x1 = conv1(_conv_mm(x1, p["e11"]["g"], p["e11"]["root"]), p["e11"])

    rest = _make_unpool_kernel()(x1, ups2d)
    xup = jnp.concatenate(
        [x1[:N1], rest, jnp.zeros((NP0 - N0, 128), _f32)], axis=0)
    x = conv0(_conv_mm2(xup, skip0, p["d00"]["g"], p["d00"]["root"], 64),
              p["d00"])
    x = conv0(_conv_mm(x, p["d01"]["g"], p["d01"]["root"]), p["d01"])

    out = _fc_head(x, p["fc_w"].astype(_f32), p["fc_b"].astype(_f32))
    return out[:N0]


# speedup vs baseline: 1.0221x; 1.0221x over previous
import functools

import jax
import jax.numpy as jnp
from jax import lax
from jax.experimental import pallas as pl
from jax.experimental.pallas import tpu as pltpu
from jax.experimental.pallas import tpu_sc as plsc

N0 = 40962
E0 = 245760
N1 = 10242
E1 = 61440
K = 3

NP0 = 43008
NP1 = 12288
NC = 2
NS = 16
NW = NC * NS
BM = 256

CAP0 = 8960
CAP1 = 2560

_f32 = jnp.float32
_i32 = jnp.int32


def _chunks(total):
    if total < 128:
        assert total % 8 == 0
        return [(0, total)]
    out = [(b, 128) for b in range(0, total - 127, 128)]
    if total % 128:
        out.append((total - 128, 128))
    return out


def _sc_mesh():
    return plsc.VectorSubcoreMesh(core_axis_name="c", subcore_axis_name="s",
                                  num_cores=NC, num_subcores=NS)


_SC_PARAMS = pltpu.CompilerParams(needs_layout_passes=False)


def _route_body(e, cap, rng,
                dst2d, src2d, px2d, py2d, z128,
                srcr, dlr, pxr, pyr, deg,
                dbd, dbs, dbx, dby, srcb, dlb, pxb, pyb, degacc, sem):
    c = lax.axis_index("c")
    s = lax.axis_index("s")
    wid = c * NS + s
    lo = wid * rng
    iota = lax.iota(_i32, 16)
    ones = jnp.ones((16,), _f32)
    lane0 = iota < 1
    zero16 = jnp.full((16,), 0, _i32)
    zf = jnp.zeros((16,), _f32)
    nch = cap // 128

    def pre(i, _):
        r = lax.shift_right_logical(i, 3)
        sl = pl.ds((i & 7) * 16, 16)
        srcb[r, sl] = zero16
        dlb[r, sl] = jnp.full((16,), rng, _i32)
        pxb[r, sl] = zf
        pyb[r, sl] = zf
        return 0

    lax.fori_loop(0, nch * 8, pre, 0)
    for b, sz in _chunks(rng // 8 + 8):
        pltpu.sync_copy(z128.at[pl.ds(0, sz)], degacc.at[pl.ds(b, sz)])

    def macro(mc, off):
        d1 = pltpu.async_copy(dst2d.at[pl.ds(mc * 16, 16)], dbd, sem)
        d2 = pltpu.async_copy(src2d.at[pl.ds(mc * 16, 16)], dbs, sem)
        d3 = pltpu.async_copy(px2d.at[pl.ds(mc * 16, 16)], dbx, sem)
        d4 = pltpu.async_copy(py2d.at[pl.ds(mc * 16, 16)], dby, sem)
        d1.wait()
        d2.wait()
        d3.wait()
        d4.wait()
        for r in range(16):
            for l in range(8):
                sl = pl.ds(l * 16, 16)
                dl = dbd[r, sl] - lo
                m = (dl >= 0) & (dl < rng)
                mi = m.astype(_i32)
                idx = jnp.minimum(off + plsc.cumsum(mi) - 1, cap - 1)
                row = lax.shift_right_logical(idx, 7)
                col = idx & 127
                plsc.store_scatter(dlb, [row, col], dl, mask=m)
                plsc.store_scatter(srcb, [row, col], dbs[r, sl], mask=m)
                plsc.store_scatter(pxb, [row, col], dbx[r, sl], mask=m)
                plsc.store_scatter(pyb, [row, col], dby[r, sl], mask=m)
                off = off + jnp.sum(mi)
        return off

    lax.fori_loop(0, e // 2048, macro, jnp.int32(0))

    def chunk(cj, _):
        def edge(q, _):
            for u in range(4):
                ei = q * 4 + u
                dls = plsc.load_gather(dlb, [zero16 + cj, zero16 + ei])
                plsc.addupdate_scatter(
                    degacc,
                    [lax.shift_right_logical(dls, 3), (dls & 7) * 16 + iota],
                    ones, mask=lane0)
            return 0

        lax.fori_loop(0, 32, edge, 0)
        return 0

    lax.fori_loop(0, nch, chunk, 0)

    pltpu.sync_copy(srcb, srcr.at[wid])
    pltpu.sync_copy(dlb, dlr.at[wid])
    pltpu.sync_copy(pxb, pxr.at[wid])
    pltpu.sync_copy(pyb, pyr.at[wid])
    pltpu.sync_copy(degacc.at[pl.ds(0, rng // 8)],
                    deg.at[pl.ds(wid * (rng // 8), rng // 8)])


def _make_route_kernel(e, np_, cap):
    rng = np_ // NW
    body = functools.partial(_route_body, e, cap, rng)
    slab_i = jax.ShapeDtypeStruct((NW, cap // 128, 128), _i32)
    slab_f = jax.ShapeDtypeStruct((NW, cap // 128, 128), _f32)
    return pl.kernel(
        body,
        out_type=(slab_i, slab_i, slab_f, slab_f,
                  jax.ShapeDtypeStruct((np_ // 8, 128), _f32)),
        mesh=_sc_mesh(),
        scratch_types=[
            pltpu.VMEM((16, 128), _i32),
            pltpu.VMEM((16, 128), _i32),
            pltpu.VMEM((16, 128), _f32),
            pltpu.VMEM((16, 128), _f32),
            pltpu.VMEM((cap // 128, 128), _i32),
            pltpu.VMEM((cap // 128, 128), _i32),
            pltpu.VMEM((cap // 128, 128), _f32),
            pltpu.VMEM((cap // 128, 128), _f32),
            pltpu.VMEM((rng // 8 + 8, 128), _f32),
            pltpu.SemaphoreType.DMA,
        ],
        compiler_params=_SC_PARAMS,
    )


def _edge_body(cap, rng, fout,
               xg, srcr, dlr, pxr, pyr, consts, zacc, out,
               src_s, dl_s, px_s, py_s, w_v, cv, xja, xjb, acc, sema, semb):
    c = lax.axis_index("c")
    s = lax.axis_index("s")
    wid = c * NS + s
    iota = lax.iota(_i32, 16)
    zero16 = jnp.full((16,), 0, _i32)

    p = 128 // fout
    psh = p.bit_length() - 1

    pltpu.sync_copy(srcr.at[wid], src_s)
    pltpu.sync_copy(dlr.at[wid], dl_s)
    pltpu.sync_copy(pxr.at[wid], px_s)
    pltpu.sync_copy(pyr.at[wid], py_s)
    pltpu.sync_copy(consts, cv)
    for b, sz in _chunks(rng // p + 8):
        pltpu.sync_copy(zacc.at[pl.ds(0, sz)], acc.at[pl.ds(b, sz)])

    def compute(cj, xj):
        def group(g, _):
            px = px_s[cj, pl.ds(g * 16, 16)]
            py = py_s[cj, pl.ds(g * 16, 16)]
            for k in range(K):
                dx = px - cv[k, :]
                dy = py - cv[K + k, :]
                w_v[k, pl.ds(g * 16, 16)] = jnp.exp(
                    dx * dx * cv[2 * K + k, :] + dy * dy * cv[3 * K + k, :])
            return 0

        lax.fori_loop(0, 8, group, 0)

        def edge(q, _):
            for u in range(4):
                ei = q * 4 + u
                dls = plsc.load_gather(dl_s, [zero16 + cj, zero16 + ei])
                w0 = plsc.load_gather(w_v, [zero16, zero16 + ei])
                w1 = plsc.load_gather(w_v, [zero16 + 1, zero16 + ei])
                w2 = plsc.load_gather(w_v, [zero16 + 2, zero16 + ei])
                row = lax.shift_right_logical(dls, psh)
                lbase = (dls & (p - 1)) * fout
                for fb in range(fout // 16):
                    sl = fb * 16
                    m = (xj[ei, pl.ds(sl, 16)] * w0
                         + xj[ei, pl.ds(fout + sl, 16)] * w1
                         + xj[ei, pl.ds(2 * fout + sl, 16)] * w2)
                    plsc.addupdate_scatter(acc, [row, lbase + sl + iota], m)
            return 0

        lax.fori_loop(0, 32, edge, 0)

    def pair(j, _):
        ca = pltpu.async_copy(xg.at[src_s.at[2 * j]], xja, sema)
        cb = pltpu.async_copy(xg.at[src_s.at[2 * j + 1]], xjb, semb)
        ca.wait()
        compute(2 * j, xja)
        cb.wait()
        compute(2 * j + 1, xjb)
        return 0

    lax.fori_loop(0, cap // 256, pair, 0)

    pltpu.sync_copy(acc.at[pl.ds(0, rng // p)],
                    out.at[pl.ds(wid * (rng // p), rng // p)])


def _make_edge_kernel(np_, cap, fout):
    rng = np_ // NW
    p = 128 // fout
    body = functools.partial(_edge_body, cap, rng, fout)
    return pl.kernel(
        body,
        out_type=jax.ShapeDtypeStruct((np_ // p, 128), _f32),
        mesh=_sc_mesh(),
        scratch_types=[
            pltpu.VMEM((cap // 128, 128), _i32),
            pltpu.VMEM((cap // 128, 128), _i32),
            pltpu.VMEM((cap // 128, 128), _f32),
            pltpu.VMEM((cap // 128, 128), _f32),
            pltpu.VMEM((K, 128), _f32),
            pltpu.VMEM((4 * K, 16), _f32),
            pltpu.VMEM((128, (K + 1) * fout), _f32),
            pltpu.VMEM((128, (K + 1) * fout), _f32),
            pltpu.VMEM((rng // p + 8, 128), _f32),
            pltpu.SemaphoreType.DMA,
            pltpu.SemaphoreType.DMA,
        ],
        compiler_params=_SC_PARAMS,
    )


def _pool_body(x, nbr2d, out, nbr_v, bufa, bufb, sem):
    c = lax.axis_index("c")
    s = lax.axis_index("s")
    wid = c * NS + s
    nch = NP1 // 128
    iota = lax.iota(_i32, 16)
    seven = jnp.full((16,), 7, _i32)

    for t in range((nch + NW - 1) // NW):
        ch = wid + t * NW

        @pl.when(ch < nch)
        def _():
            ridx = lax.rem(iota, seven) * nch + ch
            pltpu.async_copy(nbr2d.at[ridx], nbr_v, sem).wait()
            pltpu.async_copy(x.at[nbr_v.at[0]], bufa, sem).wait()

            def fold(jj):
                pltpu.async_copy(x.at[nbr_v.at[jj]], bufb, sem).wait()

                def rstep(r, _):
                    for fb in range(2):
                        sl = pl.ds(fb * 16, 16)
                        bufa[r, sl] = jnp.maximum(bufa[r, sl], bufb[r, sl])
                    return 0

                lax.fori_loop(0, 128, rstep, 0)

            for jj in range(1, 7):
                fold(jj)
            pltpu.sync_copy(bufa, out.at[pl.ds(ch * 128, 128)])


def _make_pool_kernel():
    return pl.kernel(
        _pool_body,
        out_type=jax.ShapeDtypeStruct((NP1, 128), _f32),
        mesh=_sc_mesh(),
        scratch_types=[
            pltpu.VMEM((16, 128), _i32),
            pltpu.VMEM((128, 128), _f32),
            pltpu.VMEM((128, 128), _f32),
            pltpu.SemaphoreType.DMA,
        ],
        compiler_params=_SC_PARAMS,
    )


def _unpool_body(x1, ups2d, rest, uv, bufa, bufb, sem):
    c = lax.axis_index("c")
    s = lax.axis_index("s")
    wid = c * NS + s
    nmean = (N0 - N1) // 128
    iota = lax.iota(_i32, 16)

    for t in range((nmean + NW - 1) // NW):
        ch = wid + t * NW

        @pl.when(ch < nmean)
        def _():
            ridx = (iota & 1) * nmean + ch
            pltpu.async_copy(ups2d.at[ridx], uv, sem).wait()
            pltpu.async_copy(x1.at[uv.at[0]], bufa, sem).wait()
            pltpu.async_copy(x1.at[uv.at[1]], bufb, sem).wait()

            def rstep(r, _):
                for fb in range(4):
                    sl = pl.ds(fb * 16, 16)
                    bufa[r, sl] = 0.5 * (bufa[r, sl] + bufb[r, sl])
                return 0

            lax.fori_loop(0, 128, rstep, 0)
            pltpu.sync_copy(bufa, rest.at[pl.ds(ch * 128, 128)])


def _make_unpool_kernel():
    return pl.kernel(
        _unpool_body,
        out_type=jax.ShapeDtypeStruct((N0 - N1, 128), _f32),
        mesh=_sc_mesh(),
        scratch_types=[
            pltpu.VMEM((16, 128), _i32),
            pltpu.VMEM((128, 128), _f32),
            pltpu.VMEM((128, 128), _f32),
            pltpu.SemaphoreType.DMA,
        ],
        compiler_params=_SC_PARAMS,
    )


def _mm_body(x_ref, w_ref, o_ref):
    o_ref[...] = jnp.dot(x_ref[...], w_ref[...], preferred_element_type=_f32)


def _row_pad(w):
    return jnp.pad(w, ((0, 128 - w.shape[0]), (0, 0)))


def _conv_mm(x, g, root):
    np_ = x.shape[0]
    w = _row_pad(jnp.concatenate([g, root], axis=1))
    kf = w.shape[1]
    return pl.pallas_call(
        _mm_body,
        grid=(np_ // BM,),
        in_specs=[
            pl.BlockSpec((BM, 128), lambda i: (i, 0)),
            pl.BlockSpec((128, kf), lambda i: (0, 0)),
        ],
        out_specs=pl.BlockSpec((BM, kf), lambda i: (i, 0)),
        out_shape=jax.ShapeDtypeStruct((np_, kf), _f32),
    )(x, w)


def _mm2_body(x1_ref, x2_ref, w1_ref, w2_ref, o_ref):
    o_ref[...] = (jnp.dot(x1_ref[...], w1_ref[...], preferred_element_type=_f32)
                  + jnp.dot(x2_ref[...], w2_ref[...],
                            preferred_element_type=_f32))


def _conv_mm2(x1, x2, g, root, f1):
    np_ = x1.shape[0]
    w = jnp.concatenate([g, root], axis=1)
    kf = w.shape[1]
    w1 = _row_pad(w[:f1])
    w2 = _row_pad(w[f1:])
    return pl.pallas_call(
        _mm2_body,
        grid=(np_ // BM,),
        in_specs=[
            pl.BlockSpec((BM, 128), lambda i: (i, 0)),
            pl.BlockSpec((BM, 128), lambda i: (i, 0)),
            pl.BlockSpec((128, kf), lambda i: (0, 0)),
            pl.BlockSpec((128, kf), lambda i: (0, 0)),
        ],
        out_specs=pl.BlockSpec((BM, kf), lambda i: (i, 0)),
        out_shape=jax.ShapeDtypeStruct((np_, kf), _f32),
    )(x1, x2, w1, w2)


def _epi_body(f, agg_ref, d_ref, xgr_ref, b_ref, o_ref):
    deg = jnp.maximum(d_ref[:, 0:1], 1.0)
    xr = xgr_ref[:, K * f:] + b_ref[0, :][None, :]
    res = jnp.maximum(agg_ref[...] / deg + xr, 0.0)
    o_ref[...] = jnp.pad(res, ((0, 0), (0, 128 - f)))


def _epilogue(agg, deg, xgr, bias):
    np_ = xgr.shape[0]
    f = bias.shape[0]
    b8 = jnp.tile(bias[None, :], (8, 1))
    return pl.pallas_call(
        functools.partial(_epi_body, f),
        grid=(np_ // BM,),
        in_specs=[
            pl.BlockSpec((BM, f), lambda i: (i, 0)),
            pl.BlockSpec((BM, 16), lambda i: (i, 0)),
            pl.BlockSpec((BM, (K + 1) * f), lambda i: (i, 0)),
            pl.BlockSpec((8, f), lambda i: (0, 0)),
        ],
        out_specs=pl.BlockSpec((BM, 128), lambda i: (i, 0)),
        out_shape=jax.ShapeDtypeStruct((np_, 128), _f32),
    )(agg, deg, xgr, b8)


def _fc_body(x_ref, w_ref, b_ref, o_ref):
    y = (jnp.dot(x_ref[...], w_ref[...], preferred_element_type=_f32)
         + b_ref[0, :][None, :])
    l0 = y[:, 0:1]
    l1 = y[:, 1:2]
    m = jnp.maximum(l0, l1)
    lse = m + jnp.log(jnp.exp(l0 - m) + jnp.exp(l1 - m))
    o_ref[...] = jnp.concatenate([l0 - lse, l1 - lse], axis=1)


def _fc_head(x, w, b):
    np_ = x.shape[0]
    wp = jnp.zeros((128, 128), _f32).at[:32, :2].set(w)
    bp = jnp.zeros((8, 128), _f32).at[0, :2].set(b)
    return pl.pallas_call(
        _fc_body,
        grid=(np_ // BM,),
        in_specs=[
            pl.BlockSpec((BM, 128), lambda i: (i, 0)),
            pl.BlockSpec((128, 128), lambda i: (0, 0)),
            pl.BlockSpec((8, 128), lambda i: (0, 0)),
        ],
        out_specs=pl.BlockSpec((BM, 2), lambda i: (i, 0)),
        out_shape=jax.ShapeDtypeStruct((np_, 2), _f32),
    )(x, wp, bp)


def _consts_for(p):
    mu = p["mu"].astype(_f32)
    sig = p["sigma"].astype(_f32)
    cc = -0.5 / (1e-14 + sig * sig)
    v = jnp.concatenate([mu[:, 0], mu[:, 1], cc[:, 0], cc[:, 1]])
    return jnp.tile(v[:, None], (1, 16))


def _edge_views(edge_index, pseudo):
    e = edge_index.shape[1]
    src2d = edge_index[0].astype(_i32).reshape(e // 128, 128)
    dst2d = edge_index[1].astype(_i32).reshape(e // 128, 128)
    px2d = pseudo[:, 0].astype(_f32).reshape(e // 128, 128)
    py2d = pseudo[:, 1].astype(_f32).reshape(e // 128, 128)
    return dst2d, src2d, px2d, py2d


def _pad_rows(a, n):
    return jnp.pad(a, ((0, n - a.shape[0]), (0, 0)))


def kernel(data, params, edge_index0, pseudo0, edge_index1, pseudo1,
           neigh_indices, upsample_indices):
    p = params
    x0 = _pad_rows(data.astype(_f32), NP0)

    ev0 = _edge_views(edge_index0, pseudo0)
    ev1 = _edge_views(edge_index1, pseudo1)

    nbr2d = jnp.pad(neigh_indices.astype(_i32), ((0, NP1 - N1), (0, 0)))
    nbr2d = nbr2d.T.reshape(7 * (NP1 // 128), 128)
    ups2d = upsample_indices.astype(_i32).T.reshape(2 * ((N0 - N1) // 128), 128)

    z128 = jnp.zeros((128, 128), _f32)

    srcr0, dlr0, pxr0, pyr0, deg0p = _make_route_kernel(E0, NP0, CAP0)(
        *ev0, z128)
    srcr1, dlr1, pxr1, pyr1, deg1p = _make_route_kernel(E1, NP1, CAP1)(
        *ev1, z128)
    deg0 = deg0p.reshape(NP0, 16)
    deg1 = deg1p.reshape(NP1, 16)

    edge0 = _make_edge_kernel(NP0, CAP0, 32)
    edge1 = _make_edge_kernel(NP1, CAP1, 64)

    def conv0(xgr, cp):
        agg = edge0(xgr, srcr0, dlr0, pxr0, pyr0,
                    _consts_for(cp), z128).reshape(NP0, 32)
        return _epilogue(agg, deg0, xgr, cp["bias"])

    def conv1(xgr, cp):
        agg = edge1(xgr, srcr1, dlr1, pxr1, pyr1,
                    _consts_for(cp), z128).reshape(NP1, 64)
        return _epilogue(agg, deg1, xgr, cp["bias"])

    x = conv0(_conv_mm(x0, p["e00"]["g"], p["e00"]["root"]), p["e00"])
    x = conv0(_conv_mm(x, p["e01"]["g"], p["e01"]["root"]), p["e01"])
    skip0 = x

    x1 = _make_pool_kernel()(x, nbr2d)
    x1 = conv1(_conv_mm(x1, p["e10"]["g"], p["e10"]["root"]), p["e10"])
    x1 = conv1(_conv_mm(x1, p["e11"]["g"], p["e11"]["root"]), p["e11"])

    rest = _make_unpool_kernel()(x1, ups2d)
    xup = jnp.concatenate(
        [x1[:N1], rest, jnp.zeros((NP0 - N0, 128), _f32)], axis=0)
    x = conv0(_conv_mm2(xup, skip0, p["d00"]["g"], p["d00"]["root"], 64),
              p["d00"])
    x = conv0(_conv_mm(x, p["d01"]["g"], p["d01"]["root"]), p["d01"])

    out = _fc_head(x, p["fc_w"].astype(_f32), p["fc_b"].astype(_f32))
    return out[:N0]


# speedup vs baseline: 1.0435x; 1.0209x over previous
import functools

import jax
import jax.numpy as jnp
from jax import lax
from jax.experimental import pallas as pl
from jax.experimental.pallas import tpu as pltpu
from jax.experimental.pallas import tpu_sc as plsc

N0 = 40962
E0 = 245760
N1 = 10242
E1 = 61440
K = 3

NP0 = 43008
NP1 = 12288
NC = 2
NS = 16
NW = NC * NS
BM = 256

CAP0 = 8960
CAP1 = 2560

_f32 = jnp.float32
_i32 = jnp.int32


def _chunks(total):
    if total < 128:
        assert total % 8 == 0
        return [(0, total)]
    out = [(b, 128) for b in range(0, total - 127, 128)]
    if total % 128:
        out.append((total - 128, 128))
    return out


def _sc_mesh():
    return plsc.VectorSubcoreMesh(core_axis_name="c", subcore_axis_name="s",
                                  num_cores=NC, num_subcores=NS)


_SC_PARAMS = pltpu.CompilerParams(needs_layout_passes=False)


def _route_body(e, cap, rng,
                dst2d, src2d, px2d, py2d, z128,
                srcr, dlr, pxr, pyr, deg,
                dbd, dbs, dbx, dby, srcb, dlb, pxb, pyb, degacc, sem):
    c = lax.axis_index("c")
    s = lax.axis_index("s")
    wid = c * NS + s
    lo = wid * rng
    iota = lax.iota(_i32, 16)
    ones = jnp.ones((16,), _f32)
    lane0 = iota < 1
    zero16 = jnp.full((16,), 0, _i32)
    zf = jnp.zeros((16,), _f32)
    nch = cap // 128

    def pre(i, _):
        r = lax.shift_right_logical(i, 3)
        sl = pl.ds((i & 7) * 16, 16)
        srcb[r, sl] = zero16
        dlb[r, sl] = jnp.full((16,), rng, _i32)
        pxb[r, sl] = zf
        pyb[r, sl] = zf
        return 0

    lax.fori_loop(0, nch * 8, pre, 0)
    for b, sz in _chunks(rng // 8 + 8):
        pltpu.sync_copy(z128.at[pl.ds(0, sz)], degacc.at[pl.ds(b, sz)])

    def macro(mc, off):
        d1 = pltpu.async_copy(dst2d.at[pl.ds(mc * 16, 16)], dbd, sem)
        d2 = pltpu.async_copy(src2d.at[pl.ds(mc * 16, 16)], dbs, sem)
        d3 = pltpu.async_copy(px2d.at[pl.ds(mc * 16, 16)], dbx, sem)
        d4 = pltpu.async_copy(py2d.at[pl.ds(mc * 16, 16)], dby, sem)
        d1.wait()
        d2.wait()
        d3.wait()
        d4.wait()
        for r in range(16):
            for l in range(8):
                sl = pl.ds(l * 16, 16)
                dl = dbd[r, sl] - lo
                m = (dl >= 0) & (dl < rng)
                mi = m.astype(_i32)
                idx = jnp.minimum(off + plsc.cumsum(mi) - 1, cap - 1)
                row = lax.shift_right_logical(idx, 7)
                col = idx & 127
                plsc.store_scatter(dlb, [row, col], dl, mask=m)
                plsc.store_scatter(srcb, [row, col], dbs[r, sl], mask=m)
                plsc.store_scatter(pxb, [row, col], dbx[r, sl], mask=m)
                plsc.store_scatter(pyb, [row, col], dby[r, sl], mask=m)
                off = off + jnp.sum(mi)
        return off

    lax.fori_loop(0, e // 2048, macro, jnp.int32(0))

    def chunk(cj, _):
        @plsc.parallel_loop(0, 128, 1, unroll=8)
        def edge(ei):
            dls = plsc.load_gather(dlb, [zero16 + cj, zero16 + ei])
            plsc.addupdate_scatter(
                degacc,
                [lax.shift_right_logical(dls, 3), (dls & 7) * 16 + iota],
                ones, mask=lane0)

        return 0

    lax.fori_loop(0, nch, chunk, 0)

    pltpu.sync_copy(srcb, srcr.at[wid])
    pltpu.sync_copy(dlb, dlr.at[wid])
    pltpu.sync_copy(pxb, pxr.at[wid])
    pltpu.sync_copy(pyb, pyr.at[wid])
    pltpu.sync_copy(degacc.at[pl.ds(0, rng // 8)],
                    deg.at[pl.ds(wid * (rng // 8), rng // 8)])


def _make_route_kernel(e, np_, cap):
    rng = np_ // NW
    body = functools.partial(_route_body, e, cap, rng)
    slab_i = jax.ShapeDtypeStruct((NW, cap // 128, 128), _i32)
    slab_f = jax.ShapeDtypeStruct((NW, cap // 128, 128), _f32)
    return pl.kernel(
        body,
        out_type=(slab_i, slab_i, slab_f, slab_f,
                  jax.ShapeDtypeStruct((np_ // 8, 128), _f32)),
        mesh=_sc_mesh(),
        scratch_types=[
            pltpu.VMEM((16, 128), _i32),
            pltpu.VMEM((16, 128), _i32),
            pltpu.VMEM((16, 128), _f32),
            pltpu.VMEM((16, 128), _f32),
            pltpu.VMEM((cap // 128, 128), _i32),
            pltpu.VMEM((cap // 128, 128), _i32),
            pltpu.VMEM((cap // 128, 128), _f32),
            pltpu.VMEM((cap // 128, 128), _f32),
            pltpu.VMEM((rng // 8 + 8, 128), _f32),
            pltpu.SemaphoreType.DMA,
        ],
        compiler_params=_SC_PARAMS,
    )


def _edge_body(cap, rng, fout,
               xg, srcr, dlr, pxr, pyr, consts, zacc, out,
               src_s, dl_s, px_s, py_s, w_v, cv, xja, xjb, acc, sema, semb):
    c = lax.axis_index("c")
    s = lax.axis_index("s")
    wid = c * NS + s
    iota = lax.iota(_i32, 16)
    zero16 = jnp.full((16,), 0, _i32)

    p = 128 // fout
    psh = p.bit_length() - 1

    pltpu.sync_copy(srcr.at[wid], src_s)
    pltpu.sync_copy(dlr.at[wid], dl_s)
    pltpu.sync_copy(pxr.at[wid], px_s)
    pltpu.sync_copy(pyr.at[wid], py_s)
    pltpu.sync_copy(consts, cv)
    for b, sz in _chunks(rng // p + 8):
        pltpu.sync_copy(zacc.at[pl.ds(0, sz)], acc.at[pl.ds(b, sz)])

    def compute(cj, xj):
        def group(g, _):
            px = px_s[cj, pl.ds(g * 16, 16)]
            py = py_s[cj, pl.ds(g * 16, 16)]
            for k in range(K):
                dx = px - cv[k, :]
                dy = py - cv[K + k, :]
                w_v[k, pl.ds(g * 16, 16)] = jnp.exp(
                    dx * dx * cv[2 * K + k, :] + dy * dy * cv[3 * K + k, :])
            return 0

        lax.fori_loop(0, 8, group, 0)

        @plsc.parallel_loop(0, 128, 1, unroll=8)
        def edge(ei):
            dls = plsc.load_gather(dl_s, [zero16 + cj, zero16 + ei])
            w0 = plsc.load_gather(w_v, [zero16, zero16 + ei])
            w1 = plsc.load_gather(w_v, [zero16 + 1, zero16 + ei])
            w2 = plsc.load_gather(w_v, [zero16 + 2, zero16 + ei])
            row = lax.shift_right_logical(dls, psh)
            lbase = (dls & (p - 1)) * fout
            for fb in range(fout // 16):
                sl = fb * 16
                m = (xj[ei, pl.ds(sl, 16)] * w0
                     + xj[ei, pl.ds(fout + sl, 16)] * w1
                     + xj[ei, pl.ds(2 * fout + sl, 16)] * w2)
                plsc.addupdate_scatter(acc, [row, lbase + sl + iota], m)

    def pair(j, _):
        ca = pltpu.async_copy(xg.at[src_s.at[2 * j]], xja, sema)
        cb = pltpu.async_copy(xg.at[src_s.at[2 * j + 1]], xjb, semb)
        ca.wait()
        compute(2 * j, xja)
        cb.wait()
        compute(2 * j + 1, xjb)
        return 0

    lax.fori_loop(0, cap // 256, pair, 0)

    pltpu.sync_copy(acc.at[pl.ds(0, rng // p)],
                    out.at[pl.ds(wid * (rng // p), rng // p)])


def _make_edge_kernel(np_, cap, fout):
    rng = np_ // NW
    p = 128 // fout
    body = functools.partial(_edge_body, cap, rng, fout)
    return pl.kernel(
        body,
        out_type=jax.ShapeDtypeStruct((np_ // p, 128), _f32),
        mesh=_sc_mesh(),
        scratch_types=[
            pltpu.VMEM((cap // 128, 128), _i32),
            pltpu.VMEM((cap // 128, 128), _i32),
            pltpu.VMEM((cap // 128, 128), _f32),
            pltpu.VMEM((cap // 128, 128), _f32),
            pltpu.VMEM((K, 128), _f32),
            pltpu.VMEM((4 * K, 16), _f32),
            pltpu.VMEM((128, (K + 1) * fout), _f32),
            pltpu.VMEM((128, (K + 1) * fout), _f32),
            pltpu.VMEM((rng // p + 8, 128), _f32),
            pltpu.SemaphoreType.DMA,
            pltpu.SemaphoreType.DMA,
        ],
        compiler_params=_SC_PARAMS,
    )


def _pool_body(x, nbr2d, out, nbr_v, bufa, bufb, sem):
    c = lax.axis_index("c")
    s = lax.axis_index("s")
    wid = c * NS + s
    nch = NP1 // 128
    iota = lax.iota(_i32, 16)
    seven = jnp.full((16,), 7, _i32)

    for t in range((nch + NW - 1) // NW):
        ch = wid + t * NW

        @pl.when(ch < nch)
        def _():
            ridx = lax.rem(iota, seven) * nch + ch
            pltpu.async_copy(nbr2d.at[ridx], nbr_v, sem).wait()
            pltpu.async_copy(x.at[nbr_v.at[0]], bufa, sem).wait()

            def fold(jj):
                pltpu.async_copy(x.at[nbr_v.at[jj]], bufb, sem).wait()

                def rstep(r, _):
                    for fb in range(2):
                        sl = pl.ds(fb * 16, 16)
                        bufa[r, sl] = jnp.maximum(bufa[r, sl], bufb[r, sl])
                    return 0

                lax.fori_loop(0, 128, rstep, 0)

            for jj in range(1, 7):
                fold(jj)
            pltpu.sync_copy(bufa, out.at[pl.ds(ch * 128, 128)])


def _make_pool_kernel():
    return pl.kernel(
        _pool_body,
        out_type=jax.ShapeDtypeStruct((NP1, 128), _f32),
        mesh=_sc_mesh(),
        scratch_types=[
            pltpu.VMEM((16, 128), _i32),
            pltpu.VMEM((128, 128), _f32),
            pltpu.VMEM((128, 128), _f32),
            pltpu.SemaphoreType.DMA,
        ],
        compiler_params=_SC_PARAMS,
    )


def _unpool_body(x1, ups2d, rest, uv, bufa, bufb, sem):
    c = lax.axis_index("c")
    s = lax.axis_index("s")
    wid = c * NS + s
    nmean = (N0 - N1) // 128
    iota = lax.iota(_i32, 16)

    for t in range((nmean + NW - 1) // NW):
        ch = wid + t * NW

        @pl.when(ch < nmean)
        def _():
            ridx = (iota & 1) * nmean + ch
            pltpu.async_copy(ups2d.at[ridx], uv, sem).wait()
            pltpu.async_copy(x1.at[uv.at[0]], bufa, sem).wait()
            pltpu.async_copy(x1.at[uv.at[1]], bufb, sem).wait()

            def rstep(r, _):
                for fb in range(4):
                    sl = pl.ds(fb * 16, 16)
                    bufa[r, sl] = 0.5 * (bufa[r, sl] + bufb[r, sl])
                return 0

            lax.fori_loop(0, 128, rstep, 0)
            pltpu.sync_copy(bufa, rest.at[pl.ds(ch * 128, 128)])


def _make_unpool_kernel():
    return pl.kernel(
        _unpool_body,
        out_type=jax.ShapeDtypeStruct((N0 - N1, 128), _f32),
        mesh=_sc_mesh(),
        scratch_types=[
            pltpu.VMEM((16, 128), _i32),
            pltpu.VMEM((128, 128), _f32),
            pltpu.VMEM((128, 128), _f32),
            pltpu.SemaphoreType.DMA,
        ],
        compiler_params=_SC_PARAMS,
    )


def _mm_body(x_ref, w_ref, o_ref):
    o_ref[...] = jnp.dot(x_ref[...], w_ref[...], preferred_element_type=_f32)


def _row_pad(w):
    return jnp.pad(w, ((0, 128 - w.shape[0]), (0, 0)))


def _conv_mm(x, g, root):
    np_ = x.shape[0]
    w = _row_pad(jnp.concatenate([g, root], axis=1))
    kf = w.shape[1]
    return pl.pallas_call(
        _mm_body,
        grid=(np_ // BM,),
        in_specs=[
            pl.BlockSpec((BM, 128), lambda i: (i, 0)),
            pl.BlockSpec((128, kf), lambda i: (0, 0)),
        ],
        out_specs=pl.BlockSpec((BM, kf), lambda i: (i, 0)),
        out_shape=jax.ShapeDtypeStruct((np_, kf), _f32),
    )(x, w)


def _mm2_body(x1_ref, x2_ref, w1_ref, w2_ref, o_ref):
    o_ref[...] = (jnp.dot(x1_ref[...], w1_ref[...], preferred_element_type=_f32)
                  + jnp.dot(x2_ref[...], w2_ref[...],
                            preferred_element_type=_f32))


def _conv_mm2(x1, x2, g, root, f1):
    np_ = x1.shape[0]
    w = jnp.concatenate([g, root], axis=1)
    kf = w.shape[1]
    w1 = _row_pad(w[:f1])
    w2 = _row_pad(w[f1:])
    return pl.pallas_call(
        _mm2_body,
        grid=(np_ // BM,),
        in_specs=[
            pl.BlockSpec((BM, 128), lambda i: (i, 0)),
            pl.BlockSpec((BM, 128), lambda i: (i, 0)),
            pl.BlockSpec((128, kf), lambda i: (0, 0)),
            pl.BlockSpec((128, kf), lambda i: (0, 0)),
        ],
        out_specs=pl.BlockSpec((BM, kf), lambda i: (i, 0)),
        out_shape=jax.ShapeDtypeStruct((np_, kf), _f32),
    )(x1, x2, w1, w2)


def _epi_body(f, agg_ref, d_ref, xgr_ref, b_ref, o_ref):
    deg = jnp.maximum(d_ref[:, 0:1], 1.0)
    xr = xgr_ref[:, K * f:] + b_ref[0, :][None, :]
    res = jnp.maximum(agg_ref[...] / deg + xr, 0.0)
    o_ref[...] = jnp.pad(res, ((0, 0), (0, 128 - f)))


def _epilogue(agg, deg, xgr, bias):
    np_ = xgr.shape[0]
    f = bias.shape[0]
    b8 = jnp.tile(bias[None, :], (8, 1))
    return pl.pallas_call(
        functools.partial(_epi_body, f),
        grid=(np_ // BM,),
        in_specs=[
            pl.BlockSpec((BM, f), lambda i: (i, 0)),
            pl.BlockSpec((BM, 16), lambda i: (i, 0)),
            pl.BlockSpec((BM, (K + 1) * f), lambda i: (i, 0)),
            pl.BlockSpec((8, f), lambda i: (0, 0)),
        ],
        out_specs=pl.BlockSpec((BM, 128), lambda i: (i, 0)),
        out_shape=jax.ShapeDtypeStruct((np_, 128), _f32),
    )(agg, deg, xgr, b8)


def _fc_body(x_ref, w_ref, b_ref, o_ref):
    y = (jnp.dot(x_ref[...], w_ref[...], preferred_element_type=_f32)
         + b_ref[0, :][None, :])
    l0 = y[:, 0:1]
    l1 = y[:, 1:2]
    m = jnp.maximum(l0, l1)
    lse = m + jnp.log(jnp.exp(l0 - m) + jnp.exp(l1 - m))
    o_ref[...] = jnp.concatenate([l0 - lse, l1 - lse], axis=1)


def _fc_head(x, w, b):
    np_ = x.shape[0]
    wp = jnp.zeros((128, 128), _f32).at[:32, :2].set(w)
    bp = jnp.zeros((8, 128), _f32).at[0, :2].set(b)
    return pl.pallas_call(
        _fc_body,
        grid=(np_ // BM,),
        in_specs=[
            pl.BlockSpec((BM, 128), lambda i: (i, 0)),
            pl.BlockSpec((128, 128), lambda i: (0, 0)),
            pl.BlockSpec((8, 128), lambda i: (0, 0)),
        ],
        out_specs=pl.BlockSpec((BM, 2), lambda i: (i, 0)),
        out_shape=jax.ShapeDtypeStruct((np_, 2), _f32),
    )(x, wp, bp)


def _consts_for(p):
    mu = p["mu"].astype(_f32)
    sig = p["sigma"].astype(_f32)
    cc = -0.5 / (1e-14 + sig * sig)
    v = jnp.concatenate([mu[:, 0], mu[:, 1], cc[:, 0], cc[:, 1]])
    return jnp.tile(v[:, None], (1, 16))


def _edge_views(edge_index, pseudo):
    e = edge_index.shape[1]
    src2d = edge_index[0].astype(_i32).reshape(e // 128, 128)
    dst2d = edge_index[1].astype(_i32).reshape(e // 128, 128)
    px2d = pseudo[:, 0].astype(_f32).reshape(e // 128, 128)
    py2d = pseudo[:, 1].astype(_f32).reshape(e // 128, 128)
    return dst2d, src2d, px2d, py2d


def _pad_rows(a, n):
    return jnp.pad(a, ((0, n - a.shape[0]), (0, 0)))


def kernel(data, params, edge_index0, pseudo0, edge_index1, pseudo1,
           neigh_indices, upsample_indices):
    p = params
    x0 = _pad_rows(data.astype(_f32), NP0)

    ev0 = _edge_views(edge_index0, pseudo0)
    ev1 = _edge_views(edge_index1, pseudo1)

    nbr2d = jnp.pad(neigh_indices.astype(_i32), ((0, NP1 - N1), (0, 0)))
    nbr2d = nbr2d.T.reshape(7 * (NP1 // 128), 128)
    ups2d = upsample_indices.astype(_i32).T.reshape(2 * ((N0 - N1) // 128), 128)

    z128 = jnp.zeros((128, 128), _f32)

    srcr0, dlr0, pxr0, pyr0, deg0p = _make_route_kernel(E0, NP0, CAP0)(
        *ev0, z128)
    srcr1, dlr1, pxr1, pyr1, deg1p = _make_route_kernel(E1, NP1, CAP1)(
        *ev1, z128)
    deg0 = deg0p.reshape(NP0, 16)
    deg1 = deg1p.reshape(NP1, 16)

    edge0 = _make_edge_kernel(NP0, CAP0, 32)
    edge1 = _make_edge_kernel(NP1, CAP1, 64)

    def conv0(xgr, cp):
        agg = edge0(xgr, srcr0, dlr0, pxr0, pyr0,
                    _consts_for(cp), z128).reshape(NP0, 32)
        return _epilogue(agg, deg0, xgr, cp["bias"])

    def conv1(xgr, cp):
        agg = edge1(xgr, srcr1, dlr1, pxr1, pyr1,
                    _consts_for(cp), z128).reshape(NP1, 64)
        return _epilogue(agg, deg1, xgr, cp["bias"])

    x = conv0(_conv_mm(x0, p["e00"]["g"], p["e00"]["root"]), p["e00"])
    x = conv0(_conv_mm(x, p["e01"]["g"], p["e01"]["root"]), p["e01"])
    skip0 = x

    x1 = _make_pool_kernel()(x, nbr2d)
    x1 = conv1(_conv_mm(x1, p["e10"]["g"], p["e10"]["root"]), p["e10"])
    x1 = conv1(_conv_mm(x1, p["e11"]["g"], p["e11"]["root"]), p["e11"])

    rest = _make_unpool_kernel()(x1, ups2d)
    xup = jnp.concatenate(
        [x1[:N1], rest, jnp.zeros((NP0 - N0, 128), _f32)], axis=0)
    x = conv0(_conv_mm2(xup, skip0, p["d00"]["g"], p["d00"]["root"], 64),
              p["d00"])
    x = conv0(_conv_mm(x, p["d01"]["g"], p["d01"]["root"]), p["d01"])

    out = _fc_head(x, p["fc_w"].astype(_f32), p["fc_b"].astype(_f32))
    return out[:N0]
